# baseline (speedup 1.0000x reference)
import functools

import jax
import jax.numpy as jnp
from jax import lax
from jax.experimental import pallas as pl
from jax.experimental.pallas import tpu as pltpu
from jax.experimental.pallas import tpu_sc as plsc

N = 10000
E = 320000
D = 128
NUM_LAYERS = 4

NC = 2
NS = 16
NW = NC * NS
C = 112
CW = 90
NGR = CW // 2
R = 3
E_PAD = NW * CW * C
RPT = 624
REM = N - RPT * NS
DB = D // 16

_mesh = plsc.VectorSubcoreMesh(
    core_axis_name="c", subcore_axis_name="s", num_cores=NC, num_subcores=NS
)


def _prop_body(x_hbm, meta_hbm, w_hbm, zero_hbm, out_hbm,
               meta0, meta1, meta2, wv0, wv1, wv2, rowsv,
               gs0, gs1, gs2, ss0, ss1, ss2,
               is0, is1, is2, acc):
    cid = lax.axis_index("c")
    sid = lax.axis_index("s")
    wid = sid * NC + cid
    metab = (meta0, meta1, meta2)
    wb = (wv0, wv1, wv2)
    gsem = (gs0, gs1, gs2)
    ssem = (ss0, ss1, ss2)
    isem = (is0, is1, is2)

    r0 = wid * (CW * 2)
    r0w = wid * CW

    def issue_meta(g2, q):
        pltpu.async_copy(meta_hbm.at[pl.ds(r0 + g2 * 4, 4)], metab[q],
                         isem[q])
        pltpu.async_copy(w_hbm.at[pl.ds(r0w + g2 * 2, 2)], wb[q], isem[q])

    def wait_meta(q):
        pltpu.make_async_copy(meta_hbm.at[pl.ds(r0, 4)], metab[q],
                              isem[q]).wait()
        pltpu.make_async_copy(w_hbm.at[pl.ds(r0w, 2)], wb[q],
                              isem[q]).wait()

    def issue_gather(q, c, b):
        pltpu.async_copy(x_hbm.at[metab[q].at[c * 2]], rowsv.at[b], gsem[b])

    def wait_gather(b):
        pltpu.make_async_copy(x_hbm.at[meta0.at[0]], rowsv.at[b],
                              gsem[b]).wait()

    def issue_scatter(q, c, b):
        pltpu.async_copy(rowsv.at[b], acc.at[metab[q].at[c * 2 + 1]],
                         ssem[b], add=True)

    def wait_scatter(b):
        pltpu.make_async_copy(rowsv.at[b], acc.at[meta0.at[1]],
                              ssem[b]).wait()

    def scale_chunk(q, c, b):
        wvp = wb[q]

        def sg(g2, c2):
            wvec = wvp[c, pl.ds(g2 * 16, 16)]
            for lane in range(16):
                we = wvec[lane]
                e = g2 * 16 + lane
                for dblk in range(DB):
                    sl = pl.ds(dblk * 16, 16)
                    rowsv[b, e, sl] = rowsv[b, e, sl] * we
            return c2

        lax.fori_loop(0, C // 16, sg, None)

    pltpu.sync_copy(meta_hbm.at[pl.ds(r0, 4)], meta0)
    pltpu.sync_copy(w_hbm.at[pl.ds(r0w, 2)], wv0)
    issue_meta(1, 1)
    issue_meta(2, 2)
    issue_gather(0, 0, 0)
    issue_gather(0, 1, 1)

    for k in range(3):
        sl = pl.ds(sid * RPT + k * (RPT // 3), RPT // 3)
        pltpu.sync_copy(zero_hbm.at[sl], acc.at[sl])

    @pl.when(sid == 0)
    def _():
        sl = pl.ds(RPT * NS, REM)
        pltpu.sync_copy(zero_hbm.at[sl], acc.at[sl])

    plsc.subcore_barrier()

    def body6(j, carry):
        for i in range(6):
            k = j * 6 + i
            b = i % 3
            cpar = i % 2
            q = (i // 2) % 3
            g = k // 2

            wait_gather(b)

            @pl.when(k > 0)
            def _():
                wait_scatter((b + 2) % 3)

            if cpar == 0:
                @pl.when(jnp.logical_and(g >= 1, g + 2 < NGR))
                def _():
                    issue_meta(g + 2, (q + 2) % 3)

                @pl.when(k + 2 < CW)
                def _(q2=(q + 1) % 3, b2=(b + 2) % 3):
                    wait_meta(q2)
                    issue_gather(q2, 0, b2)
            else:
                @pl.when(k + 2 < CW)
                def _(q2=(q + 1) % 3, b2=(b + 2) % 3):
                    issue_gather(q2, 1, b2)

            scale_chunk(q, cpar, b)
            issue_scatter(q, cpar, b)
        return carry

    lax.fori_loop(0, CW // 6, body6, None)
    wait_scatter((CW - 1) % 3)

    plsc.subcore_barrier()
    r1 = sid * RPT
    pltpu.sync_copy(acc.at[pl.ds(r1, RPT)],
                    out_hbm.at[pl.ds(cid * N + r1, RPT)])

    @pl.when(sid == 0)
    def _():
        pltpu.sync_copy(acc.at[pl.ds(RPT * NS, REM)],
                        out_hbm.at[pl.ds(cid * N + RPT * NS, REM)])


_sc_propagate = pl.kernel(
    _prop_body,
    out_type=jax.ShapeDtypeStruct((NC * N, D), jnp.float32),
    mesh=_mesh,
    scratch_types=[
        pltpu.VMEM((4, C), jnp.int32),
        pltpu.VMEM((4, C), jnp.int32),
        pltpu.VMEM((4, C), jnp.int32),
        pltpu.VMEM((2, C), jnp.float32),
        pltpu.VMEM((2, C), jnp.float32),
        pltpu.VMEM((2, C), jnp.float32),
        pltpu.VMEM((R, C, D), jnp.float32),
        pltpu.SemaphoreType.DMA,
        pltpu.SemaphoreType.DMA,
        pltpu.SemaphoreType.DMA,
        pltpu.SemaphoreType.DMA,
        pltpu.SemaphoreType.DMA,
        pltpu.SemaphoreType.DMA,
        pltpu.SemaphoreType.DMA,
        pltpu.SemaphoreType.DMA,
        pltpu.SemaphoreType.DMA,
        pltpu.VMEM_SHARED((N, D), jnp.float32),
    ],
    compiler_params=pltpu.CompilerParams(use_tc_tiling_on_sc=False),
)


def _combine_body(p0_ref, p1_ref, acc_ref, x_ref, accn_ref, *, scale):
    x = p0_ref[...] + p1_ref[...]
    x_ref[...] = x
    accn_ref[...] = (acc_ref[...] + x) * scale


def _combine(p0, p1, acc, scale):
    bn = 400
    grid = N // bn
    bs = pl.BlockSpec((bn, D), lambda i: (i, 0))
    return pl.pallas_call(
        functools.partial(_combine_body, scale=scale),
        grid=(grid,),
        in_specs=[bs, bs, bs],
        out_specs=[bs, bs],
        out_shape=[
            jax.ShapeDtypeStruct((N, D), jnp.float32),
            jax.ShapeDtypeStruct((N, D), jnp.float32),
        ],
    )(p0, p1, acc)


def kernel(embeds, edge_index, edge_weight):
    row = edge_index[0]
    col = edge_index[1]
    pad = E_PAD - E
    colp = jnp.concatenate(
        [col, jnp.zeros((pad,), jnp.int32)]).reshape(NW * CW, 1, C)
    rowp = jnp.concatenate(
        [row, jnp.zeros((pad,), jnp.int32)]).reshape(NW * CW, 1, C)
    wmeta = jnp.concatenate(
        [edge_weight, jnp.zeros((pad,), jnp.float32)]).reshape(NW * CW, C)
    meta = jnp.concatenate([colp, rowp], axis=1).reshape(-1, C)
    zeros = jnp.zeros((N, D), jnp.float32)
    x = embeds
    acc = embeds
    for layer in range(NUM_LAYERS):
        p = _sc_propagate(x, meta, wmeta, zeros)
        scale = 1.0 if layer < NUM_LAYERS - 1 else 1.0 / (NUM_LAYERS + 1)
        x, acc = _combine(p[:N], p[N:], acc, scale)
    return acc


# speedup vs baseline: 1.0159x; 1.0159x over previous
import functools

import jax
import jax.numpy as jnp
from jax import lax
from jax.experimental import pallas as pl
from jax.experimental.pallas import tpu as pltpu
from jax.experimental.pallas import tpu_sc as plsc

N = 10000
E = 320000
D = 128
NUM_LAYERS = 4

NC = 2
NS = 16
NW = NC * NS
C = 112
CW = 90
NGR = CW // 2
R = 3
E_PAD = NW * CW * C
RPT = 624
REM = N - RPT * NS
DB = D // 16

_mesh = plsc.VectorSubcoreMesh(
    core_axis_name="c", subcore_axis_name="s", num_cores=NC, num_subcores=NS
)


def _prop_body(x_hbm, meta_hbm, w_hbm, zero_hbm, out_hbm,
               meta0, meta1, meta2, wv0, wv1, wv2, rowsv,
               gs0, gs1, gs2, ss0, ss1, ss2,
               is0, is1, is2, acc):
    cid = lax.axis_index("c")
    sid = lax.axis_index("s")
    wid = sid * NC + cid
    metab = (meta0, meta1, meta2)
    wb = (wv0, wv1, wv2)
    gsem = (gs0, gs1, gs2)
    ssem = (ss0, ss1, ss2)
    isem = (is0, is1, is2)

    r0 = wid * (CW * 2)
    r0w = wid * CW

    def issue_meta(g2, q):
        pltpu.async_copy(meta_hbm.at[pl.ds(r0 + g2 * 4, 4)], metab[q],
                         isem[q])
        pltpu.async_copy(w_hbm.at[pl.ds(r0w + g2 * 2, 2)], wb[q], isem[q])

    def wait_meta(q):
        pltpu.make_async_copy(meta_hbm.at[pl.ds(r0, 4)], metab[q],
                              isem[q]).wait()
        pltpu.make_async_copy(w_hbm.at[pl.ds(r0w, 2)], wb[q],
                              isem[q]).wait()

    def issue_gather(q, c, b):
        pltpu.async_copy(x_hbm.at[metab[q].at[c * 2]], rowsv.at[b], gsem[b])

    def wait_gather(b):
        pltpu.make_async_copy(x_hbm.at[meta0.at[0]], rowsv.at[b],
                              gsem[b]).wait()

    def issue_scatter(q, c, b):
        pltpu.async_copy(rowsv.at[b], acc.at[metab[q].at[c * 2 + 1]],
                         ssem[b], add=True)

    def wait_scatter(b):
        pltpu.make_async_copy(rowsv.at[b], acc.at[meta0.at[1]],
                              ssem[b]).wait()

    def scale_chunk(q, c, b):
        wvp = wb[q]

        def sg(g2, c2):
            wvec = wvp[c, pl.ds(g2 * 16, 16)]
            for lane in range(16):
                we = wvec[lane]
                e = g2 * 16 + lane
                for dblk in range(DB):
                    sl = pl.ds(dblk * 16, 16)
                    rowsv[b, e, sl] = rowsv[b, e, sl] * we
            return c2

        lax.fori_loop(0, C // 16, sg, None)

    pltpu.sync_copy(meta_hbm.at[pl.ds(r0, 4)], meta0)
    pltpu.sync_copy(w_hbm.at[pl.ds(r0w, 2)], wv0)
    issue_meta(1, 1)
    issue_meta(2, 2)
    issue_gather(0, 0, 0)
    issue_gather(0, 1, 1)

    for k in range(3):
        sl = pl.ds(sid * RPT + k * (RPT // 3), RPT // 3)
        pltpu.sync_copy(zero_hbm.at[sl], acc.at[sl])

    @pl.when(sid == 0)
    def _():
        sl = pl.ds(RPT * NS, REM)
        pltpu.sync_copy(zero_hbm.at[sl], acc.at[sl])

    plsc.subcore_barrier()

    def body6(j, carry):
        for i in range(6):
            k = j * 6 + i
            b = i % 3
            cpar = i % 2
            q = (i // 2) % 3
            g = k // 2

            wait_gather(b)
            scale_chunk(q, cpar, b)
            issue_scatter(q, cpar, b)

            @pl.when(k > 0)
            def _():
                wait_scatter((b + 2) % 3)

            if cpar == 0:
                @pl.when(jnp.logical_and(g >= 1, g + 2 < NGR))
                def _():
                    issue_meta(g + 2, (q + 2) % 3)

                @pl.when(k + 2 < CW)
                def _(q2=(q + 1) % 3, b2=(b + 2) % 3):
                    wait_meta(q2)
                    issue_gather(q2, 0, b2)
            else:
                @pl.when(k + 2 < CW)
                def _(q2=(q + 1) % 3, b2=(b + 2) % 3):
                    issue_gather(q2, 1, b2)
        return carry

    lax.fori_loop(0, CW // 6, body6, None)
    wait_scatter((CW - 1) % 3)

    plsc.subcore_barrier()
    r1 = sid * RPT
    pltpu.sync_copy(acc.at[pl.ds(r1, RPT)],
                    out_hbm.at[pl.ds(cid * N + r1, RPT)])

    @pl.when(sid == 0)
    def _():
        pltpu.sync_copy(acc.at[pl.ds(RPT * NS, REM)],
                        out_hbm.at[pl.ds(cid * N + RPT * NS, REM)])


_sc_propagate = pl.kernel(
    _prop_body,
    out_type=jax.ShapeDtypeStruct((NC * N, D), jnp.float32),
    mesh=_mesh,
    scratch_types=[
        pltpu.VMEM((4, C), jnp.int32),
        pltpu.VMEM((4, C), jnp.int32),
        pltpu.VMEM((4, C), jnp.int32),
        pltpu.VMEM((2, C), jnp.float32),
        pltpu.VMEM((2, C), jnp.float32),
        pltpu.VMEM((2, C), jnp.float32),
        pltpu.VMEM((R, C, D), jnp.float32),
        pltpu.SemaphoreType.DMA,
        pltpu.SemaphoreType.DMA,
        pltpu.SemaphoreType.DMA,
        pltpu.SemaphoreType.DMA,
        pltpu.SemaphoreType.DMA,
        pltpu.SemaphoreType.DMA,
        pltpu.SemaphoreType.DMA,
        pltpu.SemaphoreType.DMA,
        pltpu.SemaphoreType.DMA,
        pltpu.VMEM_SHARED((N, D), jnp.float32),
    ],
    compiler_params=pltpu.CompilerParams(use_tc_tiling_on_sc=False),
)


def _combine_body(p0_ref, p1_ref, acc_ref, x_ref, accn_ref, *, scale):
    x = p0_ref[...] + p1_ref[...]
    x_ref[...] = x
    accn_ref[...] = (acc_ref[...] + x) * scale


def _combine(p0, p1, acc, scale):
    bn = 400
    grid = N // bn
    bs = pl.BlockSpec((bn, D), lambda i: (i, 0))
    return pl.pallas_call(
        functools.partial(_combine_body, scale=scale),
        grid=(grid,),
        in_specs=[bs, bs, bs],
        out_specs=[bs, bs],
        out_shape=[
            jax.ShapeDtypeStruct((N, D), jnp.float32),
            jax.ShapeDtypeStruct((N, D), jnp.float32),
        ],
    )(p0, p1, acc)


def kernel(embeds, edge_index, edge_weight):
    row = edge_index[0]
    col = edge_index[1]
    pad = E_PAD - E
    colp = jnp.concatenate(
        [col, jnp.zeros((pad,), jnp.int32)]).reshape(NW * CW, 1, C)
    rowp = jnp.concatenate(
        [row, jnp.zeros((pad,), jnp.int32)]).reshape(NW * CW, 1, C)
    wmeta = jnp.concatenate(
        [edge_weight, jnp.zeros((pad,), jnp.float32)]).reshape(NW * CW, C)
    meta = jnp.concatenate([colp, rowp], axis=1).reshape(-1, C)
    zeros = jnp.zeros((N, D), jnp.float32)
    x = embeds
    acc = embeds
    for layer in range(NUM_LAYERS):
        p = _sc_propagate(x, meta, wmeta, zeros)
        scale = 1.0 if layer < NUM_LAYERS - 1 else 1.0 / (NUM_LAYERS + 1)
        x, acc = _combine(p[:N], p[N:], acc, scale)
    return acc
